# Initial kernel scaffold; baseline (speedup 1.0000x reference)
import jax
import jax.numpy as jnp
from jax.experimental import pallas as pl


def kernel(x, weight, bias, gamma, beta):
    raise NotImplementedError("write your pallas kernel here")


# speedup vs baseline: 2.0816x; 2.0816x over previous
import functools

import jax
import jax.numpy as jnp
from jax.experimental import pallas as pl
from jax.experimental.pallas import tpu as pltpu

_LANE = 128
_SUBLANE = 8
_VMEM_LIMIT = 48 * 1024 * 1024
_BLOCK_BYTES = 6 * 1024 * 1024


def _ceil_to(x, m):
    return ((x + m - 1) // m) * m


def _stats_kernel(x_ref, s_ref, sx_ref):
    b = pl.program_id(1)
    t = pl.program_id(2)

    @pl.when((b == 0) & (t == 0))
    def _():
        s_ref[...] = jnp.zeros_like(s_ref)
        sx_ref[...] = jnp.zeros_like(sx_ref)

    xs = x_ref[0]
    s_ref[0] += jax.lax.dot_general(
        xs, xs, (((1,), (1,)), ((), ())),
        preferred_element_type=jnp.float32)
    sx_ref[0] += jnp.sum(xs, axis=1, keepdims=True)


def _apply_kernel(x_ref, w_ref, scale_ref, shift_ref, o_ref, *, neg_slope):
    z = jax.lax.dot_general(
        w_ref[...], x_ref[0], (((1,), (0,)), ((), ())),
        preferred_element_type=jnp.float32)
    z = z * scale_ref[...] + shift_ref[...]
    o_ref[0] = jnp.where(z > 0, z, neg_slope * z)


def _pick_tile(cp, hwp):
    if cp * hwp * 4 <= _BLOCK_BYTES:
        return hwp, 1
    n_lane_blocks = hwp // _LANE
    nt = 2
    while n_lane_blocks % nt != 0 or cp * (hwp // nt) * 4 > _BLOCK_BYTES:
        nt += 1
    return hwp // nt, nt


def kernel(x, weight, bias, gamma, beta):
    del bias
    eps = 1e-5
    neg_slope = 0.2

    B, Cin, H, W = x.shape
    Cout = weight.shape[0]
    HW = H * W
    M = B * HW

    Cp = _ceil_to(Cin, _SUBLANE)
    Cop = _ceil_to(Cout, _SUBLANE)
    HWp = _ceil_to(HW, _LANE)

    xf = x.reshape(B, Cin, HW).astype(jnp.float32)
    if (Cp, HWp) != (Cin, HW):
        xf = jnp.pad(xf, ((0, 0), (0, Cp - Cin), (0, HWp - HW)))
    w2 = weight.reshape(Cout, Cin).astype(jnp.float32)
    if (Cop, Cp) != (Cout, Cin):
        w2 = jnp.pad(w2, ((0, Cop - Cout), (0, Cp - Cin)))

    t_hw, nt = _pick_tile(Cp, HWp)

    n_cores = 2 if B % 2 == 0 else 1
    bh = B // n_cores
    s_part, sx_part = pl.pallas_call(
        _stats_kernel,
        out_shape=(
            jax.ShapeDtypeStruct((n_cores, Cp, Cp), jnp.float32),
            jax.ShapeDtypeStruct((n_cores, Cp, 1), jnp.float32),
        ),
        grid=(n_cores, bh, nt),
        in_specs=[
            pl.BlockSpec((1, Cp, t_hw), lambda c, b, t: (c * bh + b, 0, t)),
        ],
        out_specs=(
            pl.BlockSpec((1, Cp, Cp), lambda c, b, t: (c, 0, 0)),
            pl.BlockSpec((1, Cp, 1), lambda c, b, t: (c, 0, 0)),
        ),
        compiler_params=pltpu.CompilerParams(
            dimension_semantics=("parallel", "arbitrary", "arbitrary"),
            vmem_limit_bytes=_VMEM_LIMIT),
    )(xf)

    S = jnp.sum(s_part, axis=0)
    sx = jnp.sum(sx_part, axis=0)[:, 0]
    hi = jax.lax.Precision.HIGHEST
    cnt = jnp.float32(M)
    mean = jnp.dot(w2, sx, precision=hi) / cnt
    ssy = jnp.einsum("ik,kl,il->i", w2, S, w2, precision=hi)
    var = jnp.maximum(ssy / cnt - mean * mean, 0.0)
    inv = jax.lax.rsqrt(var + eps)
    g = jnp.pad(gamma.astype(jnp.float32), (0, Cop - Cout))
    bt = jnp.pad(beta.astype(jnp.float32), (0, Cop - Cout))
    scale = (g * inv)[:, None]
    shift = (bt - mean * g * inv)[:, None]

    o = pl.pallas_call(
        functools.partial(_apply_kernel, neg_slope=neg_slope),
        out_shape=jax.ShapeDtypeStruct((B, Cop, HWp), jnp.float32),
        grid=(B, nt),
        in_specs=[
            pl.BlockSpec((1, Cp, t_hw), lambda b, t: (b, 0, t)),
            pl.BlockSpec((Cop, Cp), lambda b, t: (0, 0)),
            pl.BlockSpec((Cop, 1), lambda b, t: (0, 0)),
            pl.BlockSpec((Cop, 1), lambda b, t: (0, 0)),
        ],
        out_specs=pl.BlockSpec((1, Cop, t_hw), lambda b, t: (b, 0, t)),
        compiler_params=pltpu.CompilerParams(
            dimension_semantics=("parallel", "arbitrary"),
            vmem_limit_bytes=_VMEM_LIMIT),
    )(xf, w2, scale, shift)

    out = o[:, :Cout, :HW].reshape(B, Cout, H, W)
    return out


# speedup vs baseline: 9.7107x; 4.6650x over previous
import functools

import jax
import jax.numpy as jnp
from jax.experimental import pallas as pl
from jax.experimental.pallas import tpu as pltpu

_LANE = 128
_SUBLANE = 8
_VMEM_LIMIT = 48 * 1024 * 1024


def _ceil_to(x, m):
    return ((x + m - 1) // m) * m


def _stats_kernel(x_ref, s_ref, sx_ref):
    b = pl.program_id(1)
    t = pl.program_id(2)

    @pl.when((b == 0) & (t == 0))
    def _():
        s_ref[...] = jnp.zeros_like(s_ref)
        sx_ref[...] = jnp.zeros_like(sx_ref)

    w_blk = x_ref.shape[1]
    xs0 = x_ref[0, 0]
    acc = jax.lax.dot_general(xs0, xs0, (((1,), (1,)), ((), ())),
                              preferred_element_type=jnp.float32)
    vsum = xs0
    for w in range(1, w_blk):
        xs = x_ref[0, w]
        acc += jax.lax.dot_general(xs, xs, (((1,), (1,)), ((), ())),
                                   preferred_element_type=jnp.float32)
        vsum = vsum + xs
    s_ref[0] += acc
    sx_ref[0] += jnp.sum(vsum, axis=1, keepdims=True)


def _apply_kernel(x_ref, w_ref, scale_ref, shift_ref, o_ref, *, neg_slope):
    for w in range(x_ref.shape[1]):
        z = jax.lax.dot_general(
            w_ref[...], x_ref[0, w], (((1,), (0,)), ((), ())),
            preferred_element_type=jnp.float32)
        z = z * scale_ref[...] + shift_ref[...]
        o_ref[0, w] = jnp.where(z > 0, z, neg_slope * z)


def kernel(x, weight, bias, gamma, beta):
    del bias
    eps = 1e-5
    neg_slope = 0.2

    B, Cin, H, W = x.shape
    Cout = weight.shape[0]
    M = B * H * W

    xv = x.transpose(0, 3, 1, 2).astype(jnp.float32)

    Cp = _ceil_to(Cin, _SUBLANE)
    Cop = _ceil_to(Cout, _SUBLANE)
    Hp = _ceil_to(H, _LANE)
    if (Cp, Hp) != (Cin, H):
        xv = jnp.pad(xv, ((0, 0), (0, 0), (0, Cp - Cin), (0, Hp - H)))
    w2 = weight.reshape(Cout, Cin).astype(jnp.float32)
    if (Cop, Cp) != (Cout, Cin):
        w2 = jnp.pad(w2, ((0, Cop - Cout), (0, Cp - Cin)))

    w_blk = W
    while w_blk > 1 and (Cp * Hp * 4 * w_blk > 3 * 1024 * 1024
                         or W % w_blk != 0):
        w_blk -= 1
    nw = W // w_blk

    n_cores = 2 if B % 2 == 0 else 1
    bh = B // n_cores
    s_part, sx_part = pl.pallas_call(
        _stats_kernel,
        out_shape=(
            jax.ShapeDtypeStruct((n_cores, Cp, Cp), jnp.float32),
            jax.ShapeDtypeStruct((n_cores, Cp, 1), jnp.float32),
        ),
        grid=(n_cores, bh, nw),
        in_specs=[
            pl.BlockSpec((1, w_blk, Cp, Hp),
                         lambda c, b, t: (c * bh + b, t, 0, 0)),
        ],
        out_specs=(
            pl.BlockSpec((1, Cp, Cp), lambda c, b, t: (c, 0, 0)),
            pl.BlockSpec((1, Cp, 1), lambda c, b, t: (c, 0, 0)),
        ),
        compiler_params=pltpu.CompilerParams(
            dimension_semantics=("parallel", "arbitrary", "arbitrary"),
            vmem_limit_bytes=_VMEM_LIMIT),
    )(xv)

    S = jnp.sum(s_part, axis=0)
    sx = jnp.sum(sx_part, axis=0)[:, 0]
    hi = jax.lax.Precision.HIGHEST
    cnt = jnp.float32(M)
    mean = jnp.dot(w2, sx, precision=hi) / cnt
    ssy = jnp.einsum("ik,kl,il->i", w2, S, w2, precision=hi)
    var = jnp.maximum(ssy / cnt - mean * mean, 0.0)
    inv = jax.lax.rsqrt(var + eps)
    g = jnp.pad(gamma.astype(jnp.float32), (0, Cop - Cout))
    bt = jnp.pad(beta.astype(jnp.float32), (0, Cop - Cout))
    scale = (g * inv)[:, None]
    shift = (bt - mean * g * inv)[:, None]

    o = pl.pallas_call(
        functools.partial(_apply_kernel, neg_slope=neg_slope),
        out_shape=jax.ShapeDtypeStruct((B, W, Cop, Hp), jnp.float32),
        grid=(B, nw),
        in_specs=[
            pl.BlockSpec((1, w_blk, Cp, Hp), lambda b, t: (b, t, 0, 0)),
            pl.BlockSpec((Cop, Cp), lambda b, t: (0, 0)),
            pl.BlockSpec((Cop, 1), lambda b, t: (0, 0)),
            pl.BlockSpec((Cop, 1), lambda b, t: (0, 0)),
        ],
        out_specs=pl.BlockSpec((1, w_blk, Cop, Hp), lambda b, t: (b, t, 0, 0)),
        compiler_params=pltpu.CompilerParams(
            dimension_semantics=("parallel", "arbitrary"),
            vmem_limit_bytes=_VMEM_LIMIT),
    )(xv, w2, scale, shift)

    out = o[:, :, :Cout, :H].transpose(0, 2, 3, 1)
    return out


# speedup vs baseline: 11.3264x; 1.1664x over previous
import functools

import jax
import jax.numpy as jnp
from jax.experimental import pallas as pl
from jax.experimental.pallas import tpu as pltpu

_LANE = 128
_SUBLANE = 8
_VMEM_LIMIT = 48 * 1024 * 1024


def _ceil_to(x, m):
    return ((x + m - 1) // m) * m


def _stats_kernel(x_ref, s_ref, sx_ref):
    b = pl.program_id(1)
    t = pl.program_id(2)

    @pl.when((b == 0) & (t == 0))
    def _():
        s_ref[...] = jnp.zeros_like(s_ref)
        sx_ref[...] = jnp.zeros_like(sx_ref)

    w_blk = x_ref.shape[1]
    xs0 = x_ref[0, 0]
    acc = jax.lax.dot_general(xs0, xs0, (((1,), (1,)), ((), ())),
                              preferred_element_type=jnp.float32)
    vsum = xs0
    for w in range(1, w_blk):
        xs = x_ref[0, w]
        acc += jax.lax.dot_general(xs, xs, (((1,), (1,)), ((), ())),
                                   preferred_element_type=jnp.float32)
        vsum = vsum + xs
    s_ref[0] += acc
    sx_ref[0] += jnp.sum(vsum, axis=1, keepdims=True)


def _apply_kernel(x_ref, w_ref, s_ref, sx_ref, g_ref, b_ref, o_ref,
                  *, neg_slope, cnt, eps):
    w2 = w_ref[...]
    S = s_ref[0]
    sx = sx_ref[0]
    for c in range(1, s_ref.shape[0]):
        S = S + s_ref[c]
        sx = sx + sx_ref[c]
    mean = jax.lax.dot_general(
        w2, sx, (((1,), (0,)), ((), ())),
        preferred_element_type=jnp.float32) * (1.0 / cnt)
    t1 = jax.lax.dot_general(
        w2, S, (((1,), (0,)), ((), ())),
        preferred_element_type=jnp.float32)
    ssy = jnp.sum(t1 * w2, axis=1, keepdims=True)
    var = jnp.maximum(ssy * (1.0 / cnt) - mean * mean, 0.0)
    inv = jax.lax.rsqrt(var + eps)
    scale = g_ref[...] * inv
    shift = b_ref[...] - mean * scale

    for w in range(x_ref.shape[1]):
        z = jax.lax.dot_general(
            w2, x_ref[0, w], (((1,), (0,)), ((), ())),
            preferred_element_type=jnp.float32)
        z = z * scale + shift
        o_ref[0, w] = jnp.where(z > 0, z, neg_slope * z)


def kernel(x, weight, bias, gamma, beta):
    del bias
    eps = 1e-5
    neg_slope = 0.2

    B, Cin, H, W = x.shape
    Cout = weight.shape[0]
    M = B * H * W

    xv = x.transpose(0, 3, 1, 2).astype(jnp.float32)

    Cp = _ceil_to(Cin, _SUBLANE)
    Cop = _ceil_to(Cout, _SUBLANE)
    Hp = _ceil_to(H, _LANE)
    if (Cp, Hp) != (Cin, H):
        xv = jnp.pad(xv, ((0, 0), (0, 0), (0, Cp - Cin), (0, Hp - H)))
    w2 = weight.reshape(Cout, Cin).astype(jnp.float32)
    if (Cop, Cp) != (Cout, Cin):
        w2 = jnp.pad(w2, ((0, Cop - Cout), (0, Cp - Cin)))
    g2 = jnp.pad(gamma.astype(jnp.float32), (0, Cop - Cout))[:, None]
    b2 = jnp.pad(beta.astype(jnp.float32), (0, Cop - Cout))[:, None]

    w_blk = W
    while w_blk > 1 and (Cp * Hp * 4 * w_blk > 6 * 1024 * 1024
                         or W % w_blk != 0):
        w_blk -= 1
    nw = W // w_blk

    n_cores = 2 if B % 2 == 0 else 1
    bh = B // n_cores
    s_part, sx_part = pl.pallas_call(
        _stats_kernel,
        out_shape=(
            jax.ShapeDtypeStruct((n_cores, Cp, Cp), jnp.float32),
            jax.ShapeDtypeStruct((n_cores, Cp, 1), jnp.float32),
        ),
        grid=(n_cores, bh, nw),
        in_specs=[
            pl.BlockSpec((1, w_blk, Cp, Hp),
                         lambda c, b, t: (c * bh + b, t, 0, 0)),
        ],
        out_specs=(
            pl.BlockSpec((1, Cp, Cp), lambda c, b, t: (c, 0, 0)),
            pl.BlockSpec((1, Cp, 1), lambda c, b, t: (c, 0, 0)),
        ),
        compiler_params=pltpu.CompilerParams(
            dimension_semantics=("parallel", "arbitrary", "arbitrary"),
            vmem_limit_bytes=_VMEM_LIMIT),
    )(xv)

    o = pl.pallas_call(
        functools.partial(_apply_kernel, neg_slope=neg_slope,
                          cnt=float(M), eps=eps),
        out_shape=jax.ShapeDtypeStruct((B, W, Cop, Hp), jnp.float32),
        grid=(B, nw),
        in_specs=[
            pl.BlockSpec((1, w_blk, Cp, Hp), lambda b, t: (b, t, 0, 0)),
            pl.BlockSpec((Cop, Cp), lambda b, t: (0, 0)),
            pl.BlockSpec((n_cores, Cp, Cp), lambda b, t: (0, 0, 0)),
            pl.BlockSpec((n_cores, Cp, 1), lambda b, t: (0, 0, 0)),
            pl.BlockSpec((Cop, 1), lambda b, t: (0, 0)),
            pl.BlockSpec((Cop, 1), lambda b, t: (0, 0)),
        ],
        out_specs=pl.BlockSpec((1, w_blk, Cop, Hp), lambda b, t: (b, t, 0, 0)),
        compiler_params=pltpu.CompilerParams(
            dimension_semantics=("parallel", "arbitrary"),
            vmem_limit_bytes=_VMEM_LIMIT),
    )(xv, w2, s_part, sx_part, g2, b2)

    out = o[:, :, :Cout, :H].transpose(0, 2, 3, 1)
    return out
